# baseline (speedup 1.0000x reference)
import jax
import jax.numpy as jnp
from jax import lax
from jax.experimental import pallas as pl
from jax.experimental.pallas import tpu as pltpu
from jax.experimental.pallas import tpu_sc as plsc

VOCAB = 49408
MAX_POS = 128
D = 768
B = 1024
S = 64
C = 16
TOTAL = S + C

NC = 2
NS = 16
NW = NC * NS
BPW = B // NW
L = 16
DCH = D // L
R = 32


def _body(cbp_hbm, ids_hbm, ctx_hbm, tok_hbm, pos_hbm, out_hbm,
          cbp_v, ids_v, pidx0, dst0, tidx0, pos_vmem, buf0,
          sg0, so0):
  wid = lax.axis_index("s") * NC + lax.axis_index("c")
  base = wid * BPW

  pltpu.sync_copy(cbp_hbm.at[pl.ds(base, BPW)], cbp_v)
  pltpu.sync_copy(ids_hbm.at[pl.ds(base, BPW)], ids_v)
  pltpu.sync_copy(pos_hbm.at[pl.ds(0, TOTAL)], pos_vmem)

  iota = lax.iota(jnp.int32, L)

  def splat(x):
    return jnp.full((L,), x, jnp.int32)

  def fire_tok(i, h, buf, pidx, dst, tidx, sg):
    b = base + i
    cbp = plsc.load_gather(cbp_v, [splat(i)])
    bt = splat(b * TOTAL)
    for k in range(R // L):
      t = iota + (h * R + k * L)
      pi = jnp.where(t >= cbp, t + C, t)
      pidx[pl.ds(k * L, L)] = pi
      dst[pl.ds(k * L, L)] = pi + bt
      tidx[pl.ds(k * L, L)] = plsc.load_gather(
          ids_v, [splat(i), iota + (h * R + k * L)])
    pltpu.async_copy(tok_hbm.at[tidx], buf, sg)

  def wait_g_tok(buf, tidx, sg):
    pltpu.make_async_copy(tok_hbm.at[tidx], buf, sg).wait()

  def fire_ctx(q, buf, pidx, dst, tidx, sg):
    i0 = 2 * q
    b0 = base + i0
    cbp0 = plsc.load_gather(cbp_v, [splat(i0)])
    cbp1 = plsc.load_gather(cbp_v, [splat(i0 + 1)])
    pidx[pl.ds(0, L)] = cbp0 + iota
    pidx[pl.ds(L, L)] = cbp1 + iota
    dst[pl.ds(0, L)] = splat(b0 * TOTAL) + cbp0 + iota
    dst[pl.ds(L, L)] = splat((b0 + 1) * TOTAL) + cbp1 + iota
    tidx[pl.ds(0, L)] = splat(b0 * C) + iota
    tidx[pl.ds(L, L)] = splat(b0 * C + L) + iota
    pltpu.async_copy(ctx_hbm.at[tidx], buf, sg)

  def wait_g_ctx(buf, tidx, sg):
    pltpu.make_async_copy(ctx_hbm.at[tidx], buf, sg).wait()

  def add_rows(buf, pidx):
    def row(r, carry):
      rs = splat(r)
      prow = plsc.load_gather(pidx, [rs])
      for k in range(DCH):
        col = iota + k * L
        pv = plsc.load_gather(pos_vmem, [prow, col])
        plsc.addupdate_scatter(buf, [rs, col], pv)
      return carry
    lax.fori_loop(0, R, row, 0)

  def fire_s(buf, dst, so):
    pltpu.async_copy(buf, out_hbm.at[dst], so)

  def wait_s(buf, dst, so):
    pltpu.make_async_copy(buf, out_hbm.at[dst], so).wait()

  def phase_a(m, carry):
    for h in range(2):
      fire_tok(m, h, buf0, pidx0, dst0, tidx0, sg0)
      wait_g_tok(buf0, tidx0, sg0)
      add_rows(buf0, pidx0)
      fire_s(buf0, dst0, so0)
      wait_s(buf0, dst0, so0)
    return carry

  lax.fori_loop(0, BPW, phase_a, 0)

  def phase_b(q, carry):
    fire_ctx(q, buf0, pidx0, dst0, tidx0, sg0)
    wait_g_ctx(buf0, tidx0, sg0)
    add_rows(buf0, pidx0)
    fire_s(buf0, dst0, so0)
    wait_s(buf0, dst0, so0)
    return carry

  lax.fori_loop(0, BPW // 2, phase_b, 0)


@jax.jit
def _sc_embed(cbp, ids, ctx2, token_table, pos_table):
  mesh = plsc.VectorSubcoreMesh(
      core_axis_name="c", subcore_axis_name="s", num_cores=NC, num_subcores=NS)
  f = pl.kernel(
      _body,
      out_type=jax.ShapeDtypeStruct((B * TOTAL, D), jnp.float32),
      mesh=mesh,
      compiler_params=pltpu.CompilerParams(needs_layout_passes=False),
      scratch_types=[
          pltpu.VMEM((BPW,), jnp.int32),
          pltpu.VMEM((BPW, S), jnp.int32),
          pltpu.VMEM((R,), jnp.int32),
          pltpu.VMEM((R,), jnp.int32),
          pltpu.VMEM((R,), jnp.int32),
          pltpu.VMEM((TOTAL, D), jnp.float32),
          pltpu.VMEM((R, D), jnp.float32),
          pltpu.SemaphoreType.DMA,
          pltpu.SemaphoreType.DMA,
      ],
  )
  return f(cbp, ids, ctx2, token_table, pos_table)


def kernel(ctx_embeddings, ctx_begin_pos, input_ids, token_table, pos_table):
  ctx2 = ctx_embeddings.reshape(B * C, D)
  ids = input_ids.astype(jnp.int32)
  cbp = ctx_begin_pos.astype(jnp.int32)
  out = _sc_embed(cbp, ids, ctx2, token_table, pos_table)
  return out.reshape(B, TOTAL, D)


# speedup vs baseline: 1.0002x; 1.0002x over previous
import jax
import jax.numpy as jnp
from jax import lax
from jax.experimental import pallas as pl
from jax.experimental.pallas import tpu as pltpu
from jax.experimental.pallas import tpu_sc as plsc

VOCAB = 49408
MAX_POS = 128
D = 768
B = 1024
S = 64
C = 16
TOTAL = S + C

NC = 2
NS = 16
NW = NC * NS
BPW = B // NW
L = 16
DCH = D // L
R = 32


def _body(cbp_hbm, ids_hbm, ctx_hbm, tok_hbm, pos_hbm, out_hbm,
          cbp_v, ids_v, pidx0, dst0, tidx0, pos_vmem, buf0,
          sg0, so0):
  wid = lax.axis_index("s") * NC + lax.axis_index("c")
  base = wid * BPW

  pltpu.sync_copy(cbp_hbm.at[pl.ds(base, BPW)], cbp_v)
  pltpu.sync_copy(ids_hbm.at[pl.ds(base, BPW)], ids_v)
  pltpu.sync_copy(pos_hbm.at[pl.ds(0, TOTAL)], pos_vmem)

  iota = lax.iota(jnp.int32, L)

  def splat(x):
    return jnp.full((L,), x, jnp.int32)

  def fire_tok(i, h, buf, pidx, dst, tidx, sg):
    b = base + i
    cbp = plsc.load_gather(cbp_v, [splat(i)])
    bt = splat(b * TOTAL)
    for k in range(R // L):
      t = iota + (h * R + k * L)
      pi = jnp.where(t >= cbp, t + C, t)
      pidx[pl.ds(k * L, L)] = pi
      dst[pl.ds(k * L, L)] = pi + bt
      tidx[pl.ds(k * L, L)] = plsc.load_gather(
          ids_v, [splat(i), iota + (h * R + k * L)])
    pltpu.async_copy(tok_hbm.at[tidx], buf, sg)

  def wait_g_tok(buf, tidx, sg):
    pltpu.make_async_copy(tok_hbm.at[tidx], buf, sg).wait()

  def fire_ctx(q, buf, pidx, dst, sg):
    i0 = 2 * q
    b0 = base + i0
    cbp0 = plsc.load_gather(cbp_v, [splat(i0)])
    cbp1 = plsc.load_gather(cbp_v, [splat(i0 + 1)])
    pidx[pl.ds(0, L)] = cbp0 + iota
    pidx[pl.ds(L, L)] = cbp1 + iota
    dst[pl.ds(0, L)] = splat(b0 * TOTAL) + cbp0 + iota
    dst[pl.ds(L, L)] = splat((b0 + 1) * TOTAL) + cbp1 + iota
    pltpu.async_copy(ctx_hbm.at[pl.ds(b0 * C, 2 * C)], buf, sg)

  def wait_g_ctx(buf, sg):
    pltpu.make_async_copy(ctx_hbm.at[pl.ds(0, 2 * C)], buf, sg).wait()

  def add_rows(buf, pidx):
    def row(r, carry):
      rs = splat(r)
      prow = plsc.load_gather(pidx, [rs])
      for k in range(DCH):
        col = iota + k * L
        pv = plsc.load_gather(pos_vmem, [prow, col])
        plsc.addupdate_scatter(buf, [rs, col], pv)
      return carry
    lax.fori_loop(0, R, row, 0)

  def fire_s(buf, dst, so):
    pltpu.async_copy(buf, out_hbm.at[dst], so)

  def wait_s(buf, dst, so):
    pltpu.make_async_copy(buf, out_hbm.at[dst], so).wait()

  def phase_a(m, carry):
    for h in range(2):
      fire_tok(m, h, buf0, pidx0, dst0, tidx0, sg0)
      wait_g_tok(buf0, tidx0, sg0)
      add_rows(buf0, pidx0)
      fire_s(buf0, dst0, so0)
      wait_s(buf0, dst0, so0)
    return carry

  lax.fori_loop(0, BPW, phase_a, 0)

  def phase_b(q, carry):
    fire_ctx(q, buf0, pidx0, dst0, sg0)
    wait_g_ctx(buf0, sg0)
    add_rows(buf0, pidx0)
    fire_s(buf0, dst0, so0)
    wait_s(buf0, dst0, so0)
    return carry

  lax.fori_loop(0, BPW // 2, phase_b, 0)


@jax.jit
def _sc_embed(cbp, ids, ctx2, token_table, pos_table):
  mesh = plsc.VectorSubcoreMesh(
      core_axis_name="c", subcore_axis_name="s", num_cores=NC, num_subcores=NS)
  f = pl.kernel(
      _body,
      out_type=jax.ShapeDtypeStruct((B * TOTAL, D), jnp.float32),
      mesh=mesh,
      compiler_params=pltpu.CompilerParams(needs_layout_passes=False),
      scratch_types=[
          pltpu.VMEM((BPW,), jnp.int32),
          pltpu.VMEM((BPW, S), jnp.int32),
          pltpu.VMEM((R,), jnp.int32),
          pltpu.VMEM((R,), jnp.int32),
          pltpu.VMEM((R,), jnp.int32),
          pltpu.VMEM((TOTAL, D), jnp.float32),
          pltpu.VMEM((R, D), jnp.float32),
          pltpu.SemaphoreType.DMA,
          pltpu.SemaphoreType.DMA,
      ],
  )
  return f(cbp, ids, ctx2, token_table, pos_table)


def kernel(ctx_embeddings, ctx_begin_pos, input_ids, token_table, pos_table):
  ctx2 = ctx_embeddings.reshape(B * C, D)
  ids = input_ids.astype(jnp.int32)
  cbp = ctx_begin_pos.astype(jnp.int32)
  out = _sc_embed(cbp, ids, ctx2, token_table, pos_table)
  return out.reshape(B, TOTAL, D)


# speedup vs baseline: 1.0807x; 1.0806x over previous
import jax
import jax.numpy as jnp
from jax import lax
from jax.experimental import pallas as pl
from jax.experimental.pallas import tpu as pltpu
from jax.experimental.pallas import tpu_sc as plsc

VOCAB = 49408
MAX_POS = 128
D = 768
B = 1024
S = 64
C = 16
TOTAL = S + C

NC = 2
NS = 16
NW = NC * NS
BPW = B // NW
L = 16
DCH = D // L
R = 32


def _body(cbp_hbm, ids_hbm, ctx_hbm, tok_hbm, pos_hbm, out_hbm,
          cbp_v, ids_v, pidx0, pidx1, dst0, dst1, tidx0, tidx1,
          pos_vmem, buf0, buf1, sg0, sg1, so0, so1):
  wid = lax.axis_index("s") * NC + lax.axis_index("c")
  base = wid * BPW

  pltpu.sync_copy(cbp_hbm.at[pl.ds(base, BPW)], cbp_v)
  pltpu.sync_copy(ids_hbm.at[pl.ds(base, BPW)], ids_v)
  pltpu.sync_copy(pos_hbm.at[pl.ds(0, TOTAL)], pos_vmem)

  iota = lax.iota(jnp.int32, L)

  def splat(x):
    return jnp.full((L,), x, jnp.int32)

  def fire_tok(i, h, buf, pidx, dst, tidx, sg):
    b = base + i
    cbp = plsc.load_gather(cbp_v, [splat(i)])
    bt = splat(b * TOTAL)
    for k in range(R // L):
      t = iota + (h * R + k * L)
      pi = jnp.where(t >= cbp, t + C, t)
      pidx[pl.ds(k * L, L)] = pi
      dst[pl.ds(k * L, L)] = pi + bt
      tidx[pl.ds(k * L, L)] = plsc.load_gather(
          ids_v, [splat(i), iota + (h * R + k * L)])
    pltpu.async_copy(tok_hbm.at[tidx], buf, sg)

  def wait_g_tok(buf, tidx, sg):
    pltpu.make_async_copy(tok_hbm.at[tidx], buf, sg).wait()

  def fire_ctx(q, buf, pidx, dst, sg):
    i0 = 2 * q
    b0 = base + i0
    cbp0 = plsc.load_gather(cbp_v, [splat(i0)])
    cbp1 = plsc.load_gather(cbp_v, [splat(i0 + 1)])
    pidx[pl.ds(0, L)] = cbp0 + iota
    pidx[pl.ds(L, L)] = cbp1 + iota
    dst[pl.ds(0, L)] = splat(b0 * TOTAL) + cbp0 + iota
    dst[pl.ds(L, L)] = splat((b0 + 1) * TOTAL) + cbp1 + iota
    pltpu.async_copy(ctx_hbm.at[pl.ds(b0 * C, 2 * C)], buf, sg)

  def wait_g_ctx(buf, sg):
    pltpu.make_async_copy(ctx_hbm.at[pl.ds(0, 2 * C)], buf, sg).wait()

  def add_rows(buf, pidx):
    def row(r, carry):
      rs = splat(r)
      prow = plsc.load_gather(pidx, [rs])
      for k in range(DCH):
        col = iota + k * L
        pv = plsc.load_gather(pos_vmem, [prow, col])
        plsc.addupdate_scatter(buf, [rs, col], pv)
      return carry
    lax.fori_loop(0, R, row, 0)

  def fire_s(buf, dst, so):
    pltpu.async_copy(buf, out_hbm.at[dst], so)

  def wait_s(buf, dst, so):
    pltpu.make_async_copy(buf, out_hbm.at[dst], so).wait()

  bufs = (buf0, buf1)
  pidxs = (pidx0, pidx1)
  dsts = (dst0, dst1)
  tidxs = (tidx0, tidx1)
  sgs = (sg0, sg1)
  sos = (so0, so1)

  def phase_a(m, carry):
    for h in range(2):
      pl.when(m > 0)(lambda: wait_s(bufs[h], dsts[h], sos[h]))
      fire_tok(m, h, bufs[h], pidxs[h], dsts[h], tidxs[h], sgs[h])
      wait_g_tok(bufs[h], tidxs[h], sgs[h])
      add_rows(bufs[h], pidxs[h])
      fire_s(bufs[h], dsts[h], sos[h])
    return carry

  lax.fori_loop(0, BPW, phase_a, 0)
  wait_s(buf0, dst0, so0)
  wait_s(buf1, dst1, so1)

  def phase_b(n, carry):
    for h in range(2):
      pl.when(n > 0)(lambda: wait_s(bufs[h], dsts[h], sos[h]))
      fire_ctx(2 * n + h, bufs[h], pidxs[h], dsts[h], sgs[h])
      wait_g_ctx(bufs[h], sgs[h])
      add_rows(bufs[h], pidxs[h])
      fire_s(bufs[h], dsts[h], sos[h])
    return carry

  lax.fori_loop(0, BPW // 4, phase_b, 0)
  wait_s(buf0, dst0, so0)
  wait_s(buf1, dst1, so1)


@jax.jit
def _sc_embed(cbp, ids, ctx2, token_table, pos_table):
  mesh = plsc.VectorSubcoreMesh(
      core_axis_name="c", subcore_axis_name="s", num_cores=NC, num_subcores=NS)
  f = pl.kernel(
      _body,
      out_type=jax.ShapeDtypeStruct((B * TOTAL, D), jnp.float32),
      mesh=mesh,
      compiler_params=pltpu.CompilerParams(needs_layout_passes=False),
      scratch_types=[
          pltpu.VMEM((BPW,), jnp.int32),
          pltpu.VMEM((BPW, S), jnp.int32),
          pltpu.VMEM((R,), jnp.int32),
          pltpu.VMEM((R,), jnp.int32),
          pltpu.VMEM((R,), jnp.int32),
          pltpu.VMEM((R,), jnp.int32),
          pltpu.VMEM((R,), jnp.int32),
          pltpu.VMEM((R,), jnp.int32),
          pltpu.VMEM((TOTAL, D), jnp.float32),
          pltpu.VMEM((R, D), jnp.float32),
          pltpu.VMEM((R, D), jnp.float32),
          pltpu.SemaphoreType.DMA,
          pltpu.SemaphoreType.DMA,
          pltpu.SemaphoreType.DMA,
          pltpu.SemaphoreType.DMA,
      ],
  )
  return f(cbp, ids, ctx2, token_table, pos_table)


def kernel(ctx_embeddings, ctx_begin_pos, input_ids, token_table, pos_table):
  ctx2 = ctx_embeddings.reshape(B * C, D)
  ids = input_ids.astype(jnp.int32)
  cbp = ctx_begin_pos.astype(jnp.int32)
  out = _sc_embed(cbp, ids, ctx2, token_table, pos_table)
  return out.reshape(B, TOTAL, D)
